# baseline (speedup 1.0000x reference)
import functools

import jax
import jax.numpy as jnp
from jax import lax
from jax.experimental import pallas as pl
from jax.experimental.pallas import tpu as pltpu
from jax.experimental.pallas import tpu_sc as plsc

N_NODES = 10000
NPAD = 10240
FEAT = 128
NB = 512
NBLK = NPAD // NB
ROWB = 1024
N_EDGES = 160000
NTILES = 16
EK = 64
NCH = 158
NCHT = NCH + 2
EPAD = NTILES * NCH * EK
ROWS_PER_TILE = NPAD // NTILES
POS_FILL = 1.0e6


def _sc_mesh():
    return plsc.VectorSubcoreMesh(core_axis_name="c", subcore_axis_name="s")


def _deg_body(dst_hbm, deg_hbm, dst_v, ones_v, init_v, deg_sh):
    c = lax.axis_index("c")
    s = lax.axis_index("s")

    @pl.when(c == 0)
    def _():
        def w(i, _):
            init_v[pl.ds(i * 16, 16)] = jnp.full((16,), 1.0, jnp.float32)
            return 0
        lax.fori_loop(0, ROWS_PER_TILE // 16, w, 0)
        pltpu.sync_copy(init_v, deg_sh.at[pl.ds(s * ROWS_PER_TILE, ROWS_PER_TILE)])

        def w2(i, _):
            ones_v[pl.ds(i * 16, 16)] = jnp.full((16,), 1.0, jnp.float32)
            return 0
        lax.fori_loop(0, EK // 16, w2, 0)
        pltpu.sync_copy(dst_hbm.at[s], dst_v)
        plsc.subcore_barrier()

        def body(j, _):
            pltpu.sync_copy(ones_v, deg_sh.at[dst_v.at[j]], add=True)
            return 0
        lax.fori_loop(0, NCH, body, 0)
        plsc.subcore_barrier()
        pltpu.sync_copy(deg_sh.at[pl.ds(s * ROWS_PER_TILE, ROWS_PER_TILE)],
                        deg_hbm.at[pl.ds(s * ROWS_PER_TILE, ROWS_PER_TILE)])


def _sc_degree(dst_tiles):
    return pl.kernel(
        _deg_body,
        out_type=jax.ShapeDtypeStruct((NPAD,), jnp.float32),
        mesh=_sc_mesh(),
        scratch_types=[
            pltpu.VMEM((NCH, EK), jnp.int32),
            pltpu.VMEM((EK,), jnp.float32),
            pltpu.VMEM((ROWS_PER_TILE,), jnp.float32),
            pltpu.VMEM_SHARED((NPAD,), jnp.float32),
        ],
        name="sc_degree",
    )(dst_tiles)


def _spmm_body(y_hbm, sd_hbm, out_hbm,
               idx0, idx1, buf0, buf1, acc_sh, gsem0, gsem1, isem0, isem1):
    c = lax.axis_index("c")
    s = lax.axis_index("s")
    rows = pl.ds(s * ROWS_PER_TILE, ROWS_PER_TILE)
    pltpu.sync_copy(y_hbm.at[c].at[rows], acc_sh.at[rows])
    plsc.subcore_barrier()

    pltpu.async_copy(sd_hbm.at[s].at[0], idx0, isem0)
    pltpu.async_copy(sd_hbm.at[s].at[1], idx1, isem1)
    pltpu.make_async_copy(sd_hbm.at[s].at[0], idx0, isem0).wait()
    pltpu.async_copy(y_hbm.at[c].at[idx0.at[0]], buf0, gsem0)

    def loop(i, _):
        j0 = 2 * i
        pltpu.make_async_copy(sd_hbm.at[s].at[0], idx1, isem1).wait()
        cp1 = pltpu.async_copy(y_hbm.at[c].at[idx1.at[0]], buf1, gsem1)
        pltpu.make_async_copy(y_hbm.at[c].at[idx0.at[0]], buf0, gsem0).wait()
        pltpu.sync_copy(buf0, acc_sh.at[idx0.at[1]], add=True)
        pltpu.async_copy(sd_hbm.at[s].at[j0 + 2], idx0, isem0)
        cp1.wait()
        pltpu.sync_copy(buf1, acc_sh.at[idx1.at[1]], add=True)
        pltpu.make_async_copy(sd_hbm.at[s].at[0], idx0, isem0).wait()
        pltpu.async_copy(y_hbm.at[c].at[idx0.at[0]], buf0, gsem0)
        pltpu.async_copy(sd_hbm.at[s].at[j0 + 3], idx1, isem1)
        return 0

    lax.fori_loop(0, NCH // 2, loop, 0)
    pltpu.make_async_copy(y_hbm.at[c].at[idx0.at[0]], buf0, gsem0).wait()
    pltpu.make_async_copy(sd_hbm.at[s].at[0], idx1, isem1).wait()
    plsc.subcore_barrier()
    pltpu.sync_copy(acc_sh.at[rows], out_hbm.at[c].at[rows])


def _sc_spmm(y, sd_tiles):
    return pl.kernel(
        _spmm_body,
        out_type=jax.ShapeDtypeStruct((2, NPAD, FEAT), jnp.float32),
        mesh=_sc_mesh(),
        scratch_types=(
            [pltpu.VMEM((2, EK), jnp.int32)] * 2
            + [pltpu.VMEM((EK, FEAT), jnp.float32)] * 2
            + [pltpu.VMEM_SHARED((NPAD, FEAT), jnp.float32)]
            + [pltpu.SemaphoreType.DMA] * 4
        ),
        name="sc_spmm",
    )(y, sd_tiles)


def _pass1_body(enc_in_ref, ew1_ref, eb1_ref, ew2_ref, eb2_ref,
                xx_ref, pos_ref, deg_ref, cwp_ref,
                h_ref, s_ref, dinv_ref, posw_ref):
    j = pl.program_id(0)

    @pl.when(j == 0)
    def _():
        a = jnp.maximum(enc_in_ref[...] @ ew1_ref[...] + eb1_ref[...], 0.0)
        h_ref[...] = a @ ew2_ref[...] + eb2_ref[...]
        s_ref[...] = jnp.zeros_like(s_ref)

    pos = pos_ref[...]
    xx = xx_ref[...]
    x2 = jnp.sum(xx * xx, axis=1)
    p2 = jnp.sum(pos * pos, axis=1)
    d2 = x2[:, None] - 2.0 * (xx @ pos.T) + p2[None, :]
    e = jnp.exp(-d2)
    s_ref[...] += jnp.sum(e, axis=1)
    dinv_ref[...] = lax.rsqrt(jnp.maximum(deg_ref[...], 1.0))
    posw_ref[...] = pos @ cwp_ref[...]


def _tc_pass1(enc_in8, ew1_8, eb1, ew2, eb2, points, pos_pad, deg, conv_wp):
    grid = (NBLK,)
    full = lambda shp: pl.BlockSpec(shp, lambda j: tuple(0 for _ in shp))
    return pl.pallas_call(
        _pass1_body,
        grid=grid,
        in_specs=[
            full((2 * 512, 8)), full((8, FEAT)), full((FEAT,)),
            full((FEAT, FEAT)), full((FEAT,)),
            full((2 * 512, 2)),
            pl.BlockSpec((NB, 2), lambda j: (j, 0)),
            pl.BlockSpec((NB,), lambda j: (j,)),
            full((2, FEAT)),
        ],
        out_specs=[
            full((2 * 512, FEAT)),
            full((2 * 512,)),
            pl.BlockSpec((NB,), lambda j: (j,)),
            pl.BlockSpec((NB, FEAT), lambda j: (j, 0)),
        ],
        out_shape=[
            jax.ShapeDtypeStruct((2 * 512, FEAT), jnp.float32),
            jax.ShapeDtypeStruct((2 * 512,), jnp.float32),
            jax.ShapeDtypeStruct((NPAD,), jnp.float32),
            jax.ShapeDtypeStruct((NPAD, FEAT), jnp.float32),
        ],
    )(enc_in8, ew1_8, eb1, ew2, eb2, points, pos_pad, deg, conv_wp)


def _pass2_body(xx_ref, pos_ref, h_ref, s_ref, gx_ref):
    pos = pos_ref[...]
    xx = xx_ref[...]
    x2 = jnp.sum(xx * xx, axis=1)
    p2 = jnp.sum(pos * pos, axis=1)
    d2 = x2[:, None] - 2.0 * (xx @ pos.T) + p2[None, :]
    e = jnp.exp(-d2)
    hs = h_ref[...] / s_ref[...][:, None]
    dn = (((0,), (0,)), ((), ()))
    g0 = lax.dot_general(e[:512], hs[:512], dn)
    g1 = lax.dot_general(e[512:], hs[512:], dn)
    gx_ref[...] = jnp.stack([g0, g1], axis=0)


def _tc_pass2(points, pos_pad, h, s):
    full = lambda shp: pl.BlockSpec(shp, lambda j: tuple(0 for _ in shp))
    return pl.pallas_call(
        _pass2_body,
        grid=(NBLK,),
        in_specs=[
            full((2 * 512, 2)),
            pl.BlockSpec((NB, 2), lambda j: (j, 0)),
            full((2 * 512, FEAT)), full((2 * 512,)),
        ],
        out_specs=pl.BlockSpec((2, NB, FEAT), lambda j: (0, j, 0)),
        out_shape=jax.ShapeDtypeStruct((2, NPAD, FEAT), jnp.float32),
    )(points, pos_pad, h, s)


def _step_body(do_ln, do_y, x_ref, acc_ref, dinv_ref, posw_ref, wx_ref,
               cb_ref, lng_ref, lnb_ref, x1_ref, y_ref):
    dinv = dinv_ref[...][:, None]
    if do_ln:
        z = x_ref[0] + dinv * acc_ref[0] + cb_ref[...]
        mu = jnp.mean(z, axis=1, keepdims=True)
        zc = z - mu
        var = jnp.mean(zc * zc, axis=1, keepdims=True)
        x1 = zc * lax.rsqrt(var + 1e-5) * lng_ref[...] + lnb_ref[...]
    else:
        x1 = x_ref[0]
    x1_ref[0] = x1
    if do_y:
        y_ref[0] = dinv * (x1 @ wx_ref[...] + posw_ref[...])
    else:
        y_ref[0] = jnp.zeros((ROWB, FEAT), jnp.float32)


def _tc_step(x, acc, dinv, posw, wx, cb, lng, lnb, do_ln, do_y):
    full = lambda shp: pl.BlockSpec(shp, lambda b, i: tuple(0 for _ in shp))
    rb = lambda: pl.BlockSpec((1, ROWB, FEAT), lambda b, i: (b, i, 0))
    vb = lambda: pl.BlockSpec((ROWB,), lambda b, i: (i,))
    nb = lambda: pl.BlockSpec((ROWB, FEAT), lambda b, i: (i, 0))
    return pl.pallas_call(
        functools.partial(_step_body, do_ln, do_y),
        grid=(2, NPAD // ROWB),
        in_specs=[rb(), rb(), vb(), nb(), full((FEAT, FEAT)),
                  full((FEAT,)), full((FEAT,)), full((FEAT,))],
        out_specs=[rb(), rb()],
        out_shape=[jax.ShapeDtypeStruct((2, NPAD, FEAT), jnp.float32),
                   jax.ShapeDtypeStruct((2, NPAD, FEAT), jnp.float32)],
    )(x, acc, dinv, posw, wx, cb, lng, lnb)


def _dec_body(qq_ref, pos_ref, gx_ref, w1a_ref, w1b_ref, b1_ref,
              w2_ref, b2_ref, lat_ref, s_ref, out_ref):
    j = pl.program_id(0)

    @pl.when(j == 0)
    def _():
        lat_ref[...] = jnp.zeros_like(lat_ref)
        s_ref[...] = jnp.zeros_like(s_ref)

    pos = pos_ref[...]
    qq = qq_ref[...]
    q2 = jnp.sum(qq * qq, axis=1)
    p2 = jnp.sum(pos * pos, axis=1)
    d2 = q2[:, None] - 2.0 * (qq @ pos.T) + p2[None, :]
    e = jnp.exp(-d2)
    s_ref[...] += jnp.sum(e, axis=1)
    gx = gx_ref[...]
    l0 = e[:512] @ gx[0]
    l1 = e[512:] @ gx[1]
    lat_ref[...] += jnp.concatenate([l0, l1], axis=0)

    @pl.when(j == NBLK - 1)
    def _():
        lat = lat_ref[...] / s_ref[...][:, None]
        a = jnp.maximum(lat @ w1a_ref[...] + qq @ w1b_ref[...] + b1_ref[...], 0.0)
        out_ref[...] = a @ w2_ref[...] + b2_ref[...]


def _tc_decoder(qpts, pos_pad, gx, w1a, w1b, b1, w2_8, b2_8):
    full = lambda shp: pl.BlockSpec(shp, lambda j: tuple(0 for _ in shp))
    return pl.pallas_call(
        _dec_body,
        grid=(NBLK,),
        in_specs=[
            full((2 * 512, 2)),
            pl.BlockSpec((NB, 2), lambda j: (j, 0)),
            pl.BlockSpec((2, NB, FEAT), lambda j: (0, j, 0)),
            full((FEAT, FEAT)), full((2, FEAT)), full((FEAT,)),
            full((FEAT, 8)), full((8,)),
        ],
        out_specs=[full((2 * 512, FEAT)), full((2 * 512,)),
                   full((2 * 512, 8))],
        out_shape=[jax.ShapeDtypeStruct((2 * 512, FEAT), jnp.float32),
                   jax.ShapeDtypeStruct((2 * 512,), jnp.float32),
                   jax.ShapeDtypeStruct((2 * 512, 8), jnp.float32)],
    )(qpts, pos_pad, gx, w1a, w1b, b1, w2_8, b2_8)


def kernel(inp_x, inp_y, q, pos, edge_index, enc_W1, enc_b1, enc_W2, enc_b2,
           conv_W, conv_b, ln_g, ln_b, dec_W1, dec_b1, dec_W2, dec_b2):
    f32 = jnp.float32
    bs, P, _ = inp_x.shape

    pos_pad = jnp.full((NPAD, 2), POS_FILL, f32).at[:N_NODES].set(pos)
    src = edge_index[0].astype(jnp.int32)
    dst = edge_index[1].astype(jnp.int32)
    pad = EPAD - N_EDGES
    src_p = jnp.concatenate([src, jnp.zeros((pad,), jnp.int32)])
    dst_p = jnp.concatenate([dst, jnp.full((pad,), NPAD - 1, jnp.int32)])
    dst_tiles = dst_p.reshape(NTILES, NCH, EK)
    src_tiles = src_p.reshape(NTILES, NCH, EK)
    sd_work = jnp.stack([src_tiles, dst_tiles], axis=2)
    dummy = jnp.tile(
        jnp.stack([jnp.zeros((EK,), jnp.int32),
                   jnp.full((EK,), NPAD - 1, jnp.int32)])[None, None],
        (NTILES, NCHT - NCH, 1, 1))
    sd_tiles = jnp.concatenate([sd_work, dummy], axis=1)

    points = jnp.concatenate([inp_x[0], inp_x[1]], axis=0)
    qpts = jnp.concatenate([q[0], q[1]], axis=0)
    enc_in = jnp.concatenate([inp_x, inp_y], axis=-1).reshape(2 * P, 3)
    enc_in8 = jnp.pad(enc_in, ((0, 0), (0, 5)))
    ew1_8 = jnp.pad(enc_W1, ((0, 5), (0, 0)))
    conv_wp = conv_W[:2]
    wx = conv_W[2:]
    w1a = dec_W1[:FEAT]
    w1b = dec_W1[FEAT:]
    w2_8 = jnp.pad(dec_W2, ((0, 0), (0, 7)))
    b2_8 = jnp.pad(dec_b2, ((0, 7)))

    deg = _sc_degree(dst_tiles)

    h, s, dinv, posw = _tc_pass1(enc_in8, ew1_8, enc_b1, enc_W2, enc_b2,
                                 points, pos_pad, deg, conv_wp)
    x = _tc_pass2(points, pos_pad, h, s)

    x, y = _tc_step(x, x, dinv, posw, wx, conv_b, ln_g, ln_b,
                    do_ln=False, do_y=True)
    for t in range(4):
        acc = _sc_spmm(y, sd_tiles)
        x, y = _tc_step(x, acc, dinv, posw, wx, conv_b, ln_g, ln_b,
                        do_ln=True, do_y=(t < 3))

    _, _, out8 = _tc_decoder(qpts, pos_pad, x, w1a, w1b, dec_b1, w2_8, b2_8)
    return out8.reshape(2, 512, 8)[:, :, :1]


# speedup vs baseline: 1.4066x; 1.4066x over previous
import functools

import jax
import jax.numpy as jnp
from jax import lax
from jax.experimental import pallas as pl
from jax.experimental.pallas import tpu as pltpu
from jax.experimental.pallas import tpu_sc as plsc

N_NODES = 10000
NPAD = 10240
FEAT = 128
NB = 1024
NBLK = NPAD // NB
ROWB = 2048
N_EDGES = 160000
NTILES = 16
EK = 64
NCH = 157
EPAD = NTILES * NCH * EK
ROWS_PER_TILE = NPAD // NTILES
POS_FILL = 1.0e6


def _sc_mesh():
    return plsc.VectorSubcoreMesh(core_axis_name="c", subcore_axis_name="s")


def _deg_body(dst_hbm, deg_hbm, dst_v, ones_v, init_v, deg_sh):
    c = lax.axis_index("c")
    s = lax.axis_index("s")

    @pl.when(c == 0)
    def _():
        def w(i, _):
            init_v[pl.ds(i * 16, 16)] = jnp.full((16,), 1.0, jnp.float32)
            return 0
        lax.fori_loop(0, ROWS_PER_TILE // 16, w, 0)
        pltpu.sync_copy(init_v, deg_sh.at[pl.ds(s * ROWS_PER_TILE, ROWS_PER_TILE)])

        def w2(i, _):
            ones_v[pl.ds(i * 16, 16)] = jnp.full((16,), 1.0, jnp.float32)
            return 0
        lax.fori_loop(0, EK // 16, w2, 0)
        pltpu.sync_copy(dst_hbm.at[s], dst_v)
        plsc.subcore_barrier()

        def body(j, _):
            pltpu.sync_copy(ones_v, deg_sh.at[dst_v.at[j]], add=True)
            return 0
        lax.fori_loop(0, NCH, body, 0)
        plsc.subcore_barrier()
        pltpu.sync_copy(deg_sh.at[pl.ds(s * ROWS_PER_TILE, ROWS_PER_TILE)],
                        deg_hbm.at[pl.ds(s * ROWS_PER_TILE, ROWS_PER_TILE)])


def _sc_degree(dst_tiles):
    return pl.kernel(
        _deg_body,
        out_type=jax.ShapeDtypeStruct((NPAD,), jnp.float32),
        mesh=_sc_mesh(),
        scratch_types=[
            pltpu.VMEM((NCH, EK), jnp.int32),
            pltpu.VMEM((EK,), jnp.float32),
            pltpu.VMEM((ROWS_PER_TILE,), jnp.float32),
            pltpu.VMEM_SHARED((NPAD,), jnp.float32),
        ],
        name="sc_degree",
    )(dst_tiles)


def _spmm_body(y_hbm, sd_hbm, out_hbm,
               idx0, idx1, buf0, buf1, acc_sh, gsem0, gsem1, isem0, isem1):
    c = lax.axis_index("c")
    s = lax.axis_index("s")
    rows = pl.ds(s * ROWS_PER_TILE, ROWS_PER_TILE)
    pltpu.sync_copy(y_hbm.at[c].at[rows], acc_sh.at[rows])
    plsc.subcore_barrier()

    pltpu.async_copy(sd_hbm.at[s].at[0], idx0, isem0)
    pltpu.async_copy(sd_hbm.at[s].at[1], idx1, isem1)
    pltpu.make_async_copy(sd_hbm.at[s].at[0], idx0, isem0).wait()
    pltpu.async_copy(y_hbm.at[c].at[idx0.at[0]], buf0, gsem0)

    def loop(i, _):
        j0 = 2 * i
        pltpu.make_async_copy(sd_hbm.at[s].at[0], idx1, isem1).wait()
        cp1 = pltpu.async_copy(y_hbm.at[c].at[idx1.at[0]], buf1, gsem1)
        pltpu.make_async_copy(y_hbm.at[c].at[idx0.at[0]], buf0, gsem0).wait()
        pltpu.sync_copy(buf0, acc_sh.at[idx0.at[1]], add=True)

        @pl.when(j0 + 2 < NCH)
        def _():
            pltpu.async_copy(sd_hbm.at[s].at[j0 + 2], idx0, isem0)
        cp1.wait()
        pltpu.sync_copy(buf1, acc_sh.at[idx1.at[1]], add=True)

        @pl.when(j0 + 2 < NCH)
        def _():
            pltpu.make_async_copy(sd_hbm.at[s].at[0], idx0, isem0).wait()
            pltpu.async_copy(y_hbm.at[c].at[idx0.at[0]], buf0, gsem0)

        @pl.when(j0 + 3 < NCH)
        def _():
            pltpu.async_copy(sd_hbm.at[s].at[j0 + 3], idx1, isem1)
        return 0

    lax.fori_loop(0, NCH // 2, loop, 0)
    pltpu.make_async_copy(y_hbm.at[c].at[idx0.at[0]], buf0, gsem0).wait()
    pltpu.sync_copy(buf0, acc_sh.at[idx0.at[1]], add=True)
    plsc.subcore_barrier()
    pltpu.sync_copy(acc_sh.at[rows], out_hbm.at[c].at[rows])


def _sc_spmm(y, sd_tiles):
    return pl.kernel(
        _spmm_body,
        out_type=jax.ShapeDtypeStruct((2, NPAD, FEAT), jnp.float32),
        mesh=_sc_mesh(),
        scratch_types=(
            [pltpu.VMEM((2, EK), jnp.int32)] * 2
            + [pltpu.VMEM((EK, FEAT), jnp.float32)] * 2
            + [pltpu.VMEM_SHARED((NPAD, FEAT), jnp.float32)]
            + [pltpu.SemaphoreType.DMA] * 4
        ),
        name="sc_spmm",
    )(y, sd_tiles)


def _pass1_body(enc_in_ref, ew1_ref, eb1_ref, ew2_ref, eb2_ref,
                xx_ref, pos_ref, deg_ref, cwp_ref,
                h_ref, s_ref, dinv_ref, posw_ref):
    j = pl.program_id(0)

    @pl.when(j == 0)
    def _():
        a = jnp.maximum(enc_in_ref[...] @ ew1_ref[...] + eb1_ref[...], 0.0)
        h_ref[...] = a @ ew2_ref[...] + eb2_ref[...]
        s_ref[...] = jnp.zeros_like(s_ref)

    pos = pos_ref[...]
    xx = xx_ref[...]
    x2 = jnp.sum(xx * xx, axis=1)
    p2 = jnp.sum(pos * pos, axis=1)
    d2 = x2[:, None] - 2.0 * (xx @ pos.T) + p2[None, :]
    e = jnp.exp(-d2)
    s_ref[...] += jnp.sum(e, axis=1)
    dinv_ref[...] = lax.rsqrt(jnp.maximum(deg_ref[...], 1.0))
    posw_ref[...] = pos @ cwp_ref[...]


def _tc_pass1(enc_in8, ew1_8, eb1, ew2, eb2, points, pos_pad, deg, conv_wp):
    grid = (NBLK,)
    full = lambda shp: pl.BlockSpec(shp, lambda j: tuple(0 for _ in shp))
    return pl.pallas_call(
        _pass1_body,
        grid=grid,
        in_specs=[
            full((2 * 512, 8)), full((8, FEAT)), full((FEAT,)),
            full((FEAT, FEAT)), full((FEAT,)),
            full((2 * 512, 2)),
            pl.BlockSpec((NB, 2), lambda j: (j, 0)),
            pl.BlockSpec((NB,), lambda j: (j,)),
            full((2, FEAT)),
        ],
        out_specs=[
            full((2 * 512, FEAT)),
            full((2 * 512,)),
            pl.BlockSpec((NB,), lambda j: (j,)),
            pl.BlockSpec((NB, FEAT), lambda j: (j, 0)),
        ],
        out_shape=[
            jax.ShapeDtypeStruct((2 * 512, FEAT), jnp.float32),
            jax.ShapeDtypeStruct((2 * 512,), jnp.float32),
            jax.ShapeDtypeStruct((NPAD,), jnp.float32),
            jax.ShapeDtypeStruct((NPAD, FEAT), jnp.float32),
        ],
    )(enc_in8, ew1_8, eb1, ew2, eb2, points, pos_pad, deg, conv_wp)


def _pass2_body(xx_ref, pos_ref, h_ref, s_ref, gx_ref):
    pos = pos_ref[...]
    xx = xx_ref[...]
    x2 = jnp.sum(xx * xx, axis=1)
    p2 = jnp.sum(pos * pos, axis=1)
    d2 = x2[:, None] - 2.0 * (xx @ pos.T) + p2[None, :]
    e = jnp.exp(-d2)
    hs = h_ref[...] / s_ref[...][:, None]
    dn = (((0,), (0,)), ((), ()))
    g0 = lax.dot_general(e[:512], hs[:512], dn)
    g1 = lax.dot_general(e[512:], hs[512:], dn)
    gx_ref[...] = jnp.stack([g0, g1], axis=0)


def _tc_pass2(points, pos_pad, h, s):
    full = lambda shp: pl.BlockSpec(shp, lambda j: tuple(0 for _ in shp))
    return pl.pallas_call(
        _pass2_body,
        grid=(NBLK,),
        in_specs=[
            full((2 * 512, 2)),
            pl.BlockSpec((NB, 2), lambda j: (j, 0)),
            full((2 * 512, FEAT)), full((2 * 512,)),
        ],
        out_specs=pl.BlockSpec((2, NB, FEAT), lambda j: (0, j, 0)),
        out_shape=jax.ShapeDtypeStruct((2, NPAD, FEAT), jnp.float32),
    )(points, pos_pad, h, s)


def _step_body(do_ln, do_y, x_ref, acc_ref, dinv_ref, posw_ref, wx_ref,
               cb_ref, lng_ref, lnb_ref, x1_ref, y_ref):
    dinv = dinv_ref[...][:, None]
    if do_ln:
        z = x_ref[0] + dinv * acc_ref[0] + cb_ref[...]
        mu = jnp.mean(z, axis=1, keepdims=True)
        zc = z - mu
        var = jnp.mean(zc * zc, axis=1, keepdims=True)
        x1 = zc * lax.rsqrt(var + 1e-5) * lng_ref[...] + lnb_ref[...]
    else:
        x1 = x_ref[0]
    x1_ref[0] = x1
    if do_y:
        y_ref[0] = dinv * (x1 @ wx_ref[...] + posw_ref[...])
    else:
        y_ref[0] = jnp.zeros((ROWB, FEAT), jnp.float32)


def _tc_step(x, acc, dinv, posw, wx, cb, lng, lnb, do_ln, do_y):
    full = lambda shp: pl.BlockSpec(shp, lambda b, i: tuple(0 for _ in shp))
    rb = lambda: pl.BlockSpec((1, ROWB, FEAT), lambda b, i: (b, i, 0))
    vb = lambda: pl.BlockSpec((ROWB,), lambda b, i: (i,))
    nb = lambda: pl.BlockSpec((ROWB, FEAT), lambda b, i: (i, 0))
    return pl.pallas_call(
        functools.partial(_step_body, do_ln, do_y),
        grid=(2, NPAD // ROWB),
        in_specs=[rb(), rb(), vb(), nb(), full((FEAT, FEAT)),
                  full((FEAT,)), full((FEAT,)), full((FEAT,))],
        out_specs=[rb(), rb()],
        out_shape=[jax.ShapeDtypeStruct((2, NPAD, FEAT), jnp.float32),
                   jax.ShapeDtypeStruct((2, NPAD, FEAT), jnp.float32)],
    )(x, acc, dinv, posw, wx, cb, lng, lnb)


def _dec_body(qq_ref, pos_ref, gx_ref, w1a_ref, w1b_ref, b1_ref,
              w2_ref, b2_ref, lat_ref, s_ref, out_ref):
    j = pl.program_id(0)

    @pl.when(j == 0)
    def _():
        lat_ref[...] = jnp.zeros_like(lat_ref)
        s_ref[...] = jnp.zeros_like(s_ref)

    pos = pos_ref[...]
    qq = qq_ref[...]
    q2 = jnp.sum(qq * qq, axis=1)
    p2 = jnp.sum(pos * pos, axis=1)
    d2 = q2[:, None] - 2.0 * (qq @ pos.T) + p2[None, :]
    e = jnp.exp(-d2)
    s_ref[...] += jnp.sum(e, axis=1)
    gx = gx_ref[...]
    l0 = e[:512] @ gx[0]
    l1 = e[512:] @ gx[1]
    lat_ref[...] += jnp.concatenate([l0, l1], axis=0)

    @pl.when(j == NBLK - 1)
    def _():
        lat = lat_ref[...] / s_ref[...][:, None]
        a = jnp.maximum(lat @ w1a_ref[...] + qq @ w1b_ref[...] + b1_ref[...], 0.0)
        out_ref[...] = a @ w2_ref[...] + b2_ref[...]


def _tc_decoder(qpts, pos_pad, gx, w1a, w1b, b1, w2_8, b2_8):
    full = lambda shp: pl.BlockSpec(shp, lambda j: tuple(0 for _ in shp))
    return pl.pallas_call(
        _dec_body,
        grid=(NBLK,),
        in_specs=[
            full((2 * 512, 2)),
            pl.BlockSpec((NB, 2), lambda j: (j, 0)),
            pl.BlockSpec((2, NB, FEAT), lambda j: (0, j, 0)),
            full((FEAT, FEAT)), full((2, FEAT)), full((FEAT,)),
            full((FEAT, 8)), full((8,)),
        ],
        out_specs=[full((2 * 512, FEAT)), full((2 * 512,)),
                   full((2 * 512, 8))],
        out_shape=[jax.ShapeDtypeStruct((2 * 512, FEAT), jnp.float32),
                   jax.ShapeDtypeStruct((2 * 512,), jnp.float32),
                   jax.ShapeDtypeStruct((2 * 512, 8), jnp.float32)],
    )(qpts, pos_pad, gx, w1a, w1b, b1, w2_8, b2_8)


def kernel(inp_x, inp_y, q, pos, edge_index, enc_W1, enc_b1, enc_W2, enc_b2,
           conv_W, conv_b, ln_g, ln_b, dec_W1, dec_b1, dec_W2, dec_b2):
    f32 = jnp.float32
    bs, P, _ = inp_x.shape

    pos_pad = jnp.full((NPAD, 2), POS_FILL, f32).at[:N_NODES].set(pos)
    src = edge_index[0].astype(jnp.int32)
    dst = edge_index[1].astype(jnp.int32)
    pad = EPAD - N_EDGES
    src_p = jnp.concatenate([src, jnp.zeros((pad,), jnp.int32)])
    dst_p = jnp.concatenate([dst, jnp.full((pad,), NPAD - 1, jnp.int32)])
    dst_tiles = dst_p.reshape(NTILES, NCH, EK)
    src_tiles = src_p.reshape(NTILES, NCH, EK)
    sd_tiles = jnp.stack([src_tiles, dst_tiles], axis=2)

    points = jnp.concatenate([inp_x[0], inp_x[1]], axis=0)
    qpts = jnp.concatenate([q[0], q[1]], axis=0)
    enc_in = jnp.concatenate([inp_x, inp_y], axis=-1).reshape(2 * P, 3)
    enc_in8 = jnp.pad(enc_in, ((0, 0), (0, 5)))
    ew1_8 = jnp.pad(enc_W1, ((0, 5), (0, 0)))
    conv_wp = conv_W[:2]
    wx = conv_W[2:]
    w1a = dec_W1[:FEAT]
    w1b = dec_W1[FEAT:]
    w2_8 = jnp.pad(dec_W2, ((0, 0), (0, 7)))
    b2_8 = jnp.pad(dec_b2, ((0, 7)))

    deg = _sc_degree(dst_tiles)

    h, s, dinv, posw = _tc_pass1(enc_in8, ew1_8, enc_b1, enc_W2, enc_b2,
                                 points, pos_pad, deg, conv_wp)
    x = _tc_pass2(points, pos_pad, h, s)

    x, y = _tc_step(x, x, dinv, posw, wx, conv_b, ln_g, ln_b,
                    do_ln=False, do_y=True)
    for t in range(4):
        acc = _sc_spmm(y, sd_tiles)
        x, y = _tc_step(x, acc, dinv, posw, wx, conv_b, ln_g, ln_b,
                        do_ln=True, do_y=(t < 3))

    _, _, out8 = _tc_decoder(qpts, pos_pad, x, w1a, w1b, dec_b1, w2_8, b2_8)
    return out8.reshape(2, 512, 8)[:, :, :1]


# speedup vs baseline: 1.4293x; 1.0161x over previous
import functools

import jax
import jax.numpy as jnp
from jax import lax
from jax.experimental import pallas as pl
from jax.experimental.pallas import tpu as pltpu
from jax.experimental.pallas import tpu_sc as plsc

N_NODES = 10000
NPAD = 10240
FEAT = 128
NB = 2048
NBLK = NPAD // NB
ROWB = 2048
N_EDGES = 160000
NTILES = 16
EK = 64
NCH = 157
EPAD = NTILES * NCH * EK
ROWS_PER_TILE = NPAD // NTILES
POS_FILL = 1.0e6


def _sc_mesh():
    return plsc.VectorSubcoreMesh(core_axis_name="c", subcore_axis_name="s")


def _deg_body(dst_hbm, deg_hbm, dst_v, ones_v, init_v, deg_sh):
    c = lax.axis_index("c")
    s = lax.axis_index("s")

    @pl.when(c == 0)
    def _():
        def w(i, _):
            init_v[pl.ds(i * 16, 16)] = jnp.full((16,), 1.0, jnp.float32)
            return 0
        lax.fori_loop(0, ROWS_PER_TILE // 16, w, 0)
        pltpu.sync_copy(init_v, deg_sh.at[pl.ds(s * ROWS_PER_TILE, ROWS_PER_TILE)])

        def w2(i, _):
            ones_v[pl.ds(i * 16, 16)] = jnp.full((16,), 1.0, jnp.float32)
            return 0
        lax.fori_loop(0, EK // 16, w2, 0)
        pltpu.sync_copy(dst_hbm.at[s], dst_v)
        plsc.subcore_barrier()

        def body(j, _):
            pltpu.sync_copy(ones_v, deg_sh.at[dst_v.at[j]], add=True)
            return 0
        lax.fori_loop(0, NCH, body, 0)
        plsc.subcore_barrier()
        pltpu.sync_copy(deg_sh.at[pl.ds(s * ROWS_PER_TILE, ROWS_PER_TILE)],
                        deg_hbm.at[pl.ds(s * ROWS_PER_TILE, ROWS_PER_TILE)])


def _sc_degree(dst_tiles):
    return pl.kernel(
        _deg_body,
        out_type=jax.ShapeDtypeStruct((NPAD,), jnp.float32),
        mesh=_sc_mesh(),
        scratch_types=[
            pltpu.VMEM((NCH, EK), jnp.int32),
            pltpu.VMEM((EK,), jnp.float32),
            pltpu.VMEM((ROWS_PER_TILE,), jnp.float32),
            pltpu.VMEM_SHARED((NPAD,), jnp.float32),
        ],
        name="sc_degree",
    )(dst_tiles)


def _spmm_body(y_hbm, sd_hbm, out_hbm,
               idx0, idx1, buf0, buf1, acc_sh, gsem0, gsem1, isem0, isem1):
    c = lax.axis_index("c")
    s = lax.axis_index("s")
    rows = pl.ds(s * ROWS_PER_TILE, ROWS_PER_TILE)
    pltpu.sync_copy(y_hbm.at[c].at[rows], acc_sh.at[rows])
    plsc.subcore_barrier()

    pltpu.async_copy(sd_hbm.at[s].at[0], idx0, isem0)
    pltpu.async_copy(sd_hbm.at[s].at[1], idx1, isem1)
    pltpu.make_async_copy(sd_hbm.at[s].at[0], idx0, isem0).wait()
    pltpu.async_copy(y_hbm.at[c].at[idx0.at[0]], buf0, gsem0)

    def loop(i, _):
        j0 = 2 * i
        pltpu.make_async_copy(sd_hbm.at[s].at[0], idx1, isem1).wait()
        cp1 = pltpu.async_copy(y_hbm.at[c].at[idx1.at[0]], buf1, gsem1)
        pltpu.make_async_copy(y_hbm.at[c].at[idx0.at[0]], buf0, gsem0).wait()
        pltpu.sync_copy(buf0, acc_sh.at[idx0.at[1]], add=True)

        @pl.when(j0 + 2 < NCH)
        def _():
            pltpu.async_copy(sd_hbm.at[s].at[j0 + 2], idx0, isem0)
        cp1.wait()
        pltpu.sync_copy(buf1, acc_sh.at[idx1.at[1]], add=True)

        @pl.when(j0 + 2 < NCH)
        def _():
            pltpu.make_async_copy(sd_hbm.at[s].at[0], idx0, isem0).wait()
            pltpu.async_copy(y_hbm.at[c].at[idx0.at[0]], buf0, gsem0)

        @pl.when(j0 + 3 < NCH)
        def _():
            pltpu.async_copy(sd_hbm.at[s].at[j0 + 3], idx1, isem1)
        return 0

    lax.fori_loop(0, NCH // 2, loop, 0)
    pltpu.make_async_copy(y_hbm.at[c].at[idx0.at[0]], buf0, gsem0).wait()
    pltpu.sync_copy(buf0, acc_sh.at[idx0.at[1]], add=True)
    plsc.subcore_barrier()
    pltpu.sync_copy(acc_sh.at[rows], out_hbm.at[c].at[rows])


def _sc_spmm(y, sd_tiles):
    return pl.kernel(
        _spmm_body,
        out_type=jax.ShapeDtypeStruct((2, NPAD, FEAT), jnp.float32),
        mesh=_sc_mesh(),
        scratch_types=(
            [pltpu.VMEM((2, EK), jnp.int32)] * 2
            + [pltpu.VMEM((EK, FEAT), jnp.float32)] * 2
            + [pltpu.VMEM_SHARED((NPAD, FEAT), jnp.float32)]
            + [pltpu.SemaphoreType.DMA] * 4
        ),
        name="sc_spmm",
    )(y, sd_tiles)


def _pass1_body(enc_in_ref, ew1_ref, eb1_ref, ew2_ref, eb2_ref,
                xx_ref, pos_ref, deg_ref, cwp_ref,
                h_ref, s_ref, dinv_ref, posw_ref):
    j = pl.program_id(0)

    @pl.when(j == 0)
    def _():
        a = jnp.maximum(enc_in_ref[...] @ ew1_ref[...] + eb1_ref[...], 0.0)
        h_ref[...] = a @ ew2_ref[...] + eb2_ref[...]
        s_ref[...] = jnp.zeros_like(s_ref)

    pos = pos_ref[...]
    xx = xx_ref[...]
    x2 = jnp.sum(xx * xx, axis=1)
    p2 = jnp.sum(pos * pos, axis=1)
    d2 = x2[:, None] - 2.0 * (xx @ pos.T) + p2[None, :]
    e = jnp.exp(-d2)
    s_ref[...] += jnp.sum(e, axis=1)
    dinv_ref[...] = lax.rsqrt(jnp.maximum(deg_ref[...], 1.0))
    posw_ref[...] = pos @ cwp_ref[...]


def _tc_pass1(enc_in8, ew1_8, eb1, ew2, eb2, points, pos_pad, deg, conv_wp):
    grid = (NBLK,)
    full = lambda shp: pl.BlockSpec(shp, lambda j: tuple(0 for _ in shp))
    return pl.pallas_call(
        _pass1_body,
        grid=grid,
        in_specs=[
            full((2 * 512, 8)), full((8, FEAT)), full((FEAT,)),
            full((FEAT, FEAT)), full((FEAT,)),
            full((2 * 512, 2)),
            pl.BlockSpec((NB, 2), lambda j: (j, 0)),
            pl.BlockSpec((NB,), lambda j: (j,)),
            full((2, FEAT)),
        ],
        out_specs=[
            full((2 * 512, FEAT)),
            full((2 * 512,)),
            pl.BlockSpec((NB,), lambda j: (j,)),
            pl.BlockSpec((NB, FEAT), lambda j: (j, 0)),
        ],
        out_shape=[
            jax.ShapeDtypeStruct((2 * 512, FEAT), jnp.float32),
            jax.ShapeDtypeStruct((2 * 512,), jnp.float32),
            jax.ShapeDtypeStruct((NPAD,), jnp.float32),
            jax.ShapeDtypeStruct((NPAD, FEAT), jnp.float32),
        ],
    )(enc_in8, ew1_8, eb1, ew2, eb2, points, pos_pad, deg, conv_wp)


def _pass2_body(xx_ref, pos_ref, h_ref, s_ref, gx_ref):
    pos = pos_ref[...]
    xx = xx_ref[...]
    x2 = jnp.sum(xx * xx, axis=1)
    p2 = jnp.sum(pos * pos, axis=1)
    d2 = x2[:, None] - 2.0 * (xx @ pos.T) + p2[None, :]
    e = jnp.exp(-d2)
    hs = h_ref[...] / s_ref[...][:, None]
    dn = (((0,), (0,)), ((), ()))
    g0 = lax.dot_general(e[:512], hs[:512], dn)
    g1 = lax.dot_general(e[512:], hs[512:], dn)
    gx_ref[...] = jnp.stack([g0, g1], axis=0)


def _tc_pass2(points, pos_pad, h, s):
    full = lambda shp: pl.BlockSpec(shp, lambda j: tuple(0 for _ in shp))
    return pl.pallas_call(
        _pass2_body,
        grid=(NBLK,),
        in_specs=[
            full((2 * 512, 2)),
            pl.BlockSpec((NB, 2), lambda j: (j, 0)),
            full((2 * 512, FEAT)), full((2 * 512,)),
        ],
        out_specs=pl.BlockSpec((2, NB, FEAT), lambda j: (0, j, 0)),
        out_shape=jax.ShapeDtypeStruct((2, NPAD, FEAT), jnp.float32),
    )(points, pos_pad, h, s)


def _step_body(do_ln, do_y, x_ref, acc_ref, dinv_ref, posw_ref, wx_ref,
               cb_ref, lng_ref, lnb_ref, x1_ref, y_ref):
    dinv = dinv_ref[...][:, None]
    if do_ln:
        z = x_ref[0] + dinv * acc_ref[0] + cb_ref[...]
        mu = jnp.mean(z, axis=1, keepdims=True)
        zc = z - mu
        var = jnp.mean(zc * zc, axis=1, keepdims=True)
        x1 = zc * lax.rsqrt(var + 1e-5) * lng_ref[...] + lnb_ref[...]
    else:
        x1 = x_ref[0]
    x1_ref[0] = x1
    if do_y:
        y_ref[0] = dinv * (x1 @ wx_ref[...] + posw_ref[...])
    else:
        y_ref[0] = jnp.zeros((ROWB, FEAT), jnp.float32)


def _tc_step(x, acc, dinv, posw, wx, cb, lng, lnb, do_ln, do_y):
    full = lambda shp: pl.BlockSpec(shp, lambda b, i: tuple(0 for _ in shp))
    rb = lambda: pl.BlockSpec((1, ROWB, FEAT), lambda b, i: (b, i, 0))
    vb = lambda: pl.BlockSpec((ROWB,), lambda b, i: (i,))
    nb = lambda: pl.BlockSpec((ROWB, FEAT), lambda b, i: (i, 0))
    return pl.pallas_call(
        functools.partial(_step_body, do_ln, do_y),
        grid=(2, NPAD // ROWB),
        in_specs=[rb(), rb(), vb(), nb(), full((FEAT, FEAT)),
                  full((FEAT,)), full((FEAT,)), full((FEAT,))],
        out_specs=[rb(), rb()],
        out_shape=[jax.ShapeDtypeStruct((2, NPAD, FEAT), jnp.float32),
                   jax.ShapeDtypeStruct((2, NPAD, FEAT), jnp.float32)],
    )(x, acc, dinv, posw, wx, cb, lng, lnb)


def _dec_body(qq_ref, pos_ref, gx_ref, w1a_ref, w1b_ref, b1_ref,
              w2_ref, b2_ref, lat_ref, s_ref, out_ref):
    j = pl.program_id(0)

    @pl.when(j == 0)
    def _():
        lat_ref[...] = jnp.zeros_like(lat_ref)
        s_ref[...] = jnp.zeros_like(s_ref)

    pos = pos_ref[...]
    qq = qq_ref[...]
    q2 = jnp.sum(qq * qq, axis=1)
    p2 = jnp.sum(pos * pos, axis=1)
    d2 = q2[:, None] - 2.0 * (qq @ pos.T) + p2[None, :]
    e = jnp.exp(-d2)
    s_ref[...] += jnp.sum(e, axis=1)
    gx = gx_ref[...]
    l0 = e[:512] @ gx[0]
    l1 = e[512:] @ gx[1]
    lat_ref[...] += jnp.concatenate([l0, l1], axis=0)

    @pl.when(j == NBLK - 1)
    def _():
        lat = lat_ref[...] / s_ref[...][:, None]
        a = jnp.maximum(lat @ w1a_ref[...] + qq @ w1b_ref[...] + b1_ref[...], 0.0)
        out_ref[...] = a @ w2_ref[...] + b2_ref[...]


def _tc_decoder(qpts, pos_pad, gx, w1a, w1b, b1, w2_8, b2_8):
    full = lambda shp: pl.BlockSpec(shp, lambda j: tuple(0 for _ in shp))
    return pl.pallas_call(
        _dec_body,
        grid=(NBLK,),
        in_specs=[
            full((2 * 512, 2)),
            pl.BlockSpec((NB, 2), lambda j: (j, 0)),
            pl.BlockSpec((2, NB, FEAT), lambda j: (0, j, 0)),
            full((FEAT, FEAT)), full((2, FEAT)), full((FEAT,)),
            full((FEAT, 8)), full((8,)),
        ],
        out_specs=[full((2 * 512, FEAT)), full((2 * 512,)),
                   full((2 * 512, 8))],
        out_shape=[jax.ShapeDtypeStruct((2 * 512, FEAT), jnp.float32),
                   jax.ShapeDtypeStruct((2 * 512,), jnp.float32),
                   jax.ShapeDtypeStruct((2 * 512, 8), jnp.float32)],
    )(qpts, pos_pad, gx, w1a, w1b, b1, w2_8, b2_8)


def kernel(inp_x, inp_y, q, pos, edge_index, enc_W1, enc_b1, enc_W2, enc_b2,
           conv_W, conv_b, ln_g, ln_b, dec_W1, dec_b1, dec_W2, dec_b2):
    f32 = jnp.float32
    bs, P, _ = inp_x.shape

    pos_pad = jnp.full((NPAD, 2), POS_FILL, f32).at[:N_NODES].set(pos)
    src = edge_index[0].astype(jnp.int32)
    dst = edge_index[1].astype(jnp.int32)
    pad = EPAD - N_EDGES
    src_p = jnp.concatenate([src, jnp.zeros((pad,), jnp.int32)])
    dst_p = jnp.concatenate([dst, jnp.full((pad,), NPAD - 1, jnp.int32)])
    dst_tiles = dst_p.reshape(NTILES, NCH, EK)
    src_tiles = src_p.reshape(NTILES, NCH, EK)
    sd_tiles = jnp.stack([src_tiles, dst_tiles], axis=2)

    points = jnp.concatenate([inp_x[0], inp_x[1]], axis=0)
    qpts = jnp.concatenate([q[0], q[1]], axis=0)
    enc_in = jnp.concatenate([inp_x, inp_y], axis=-1).reshape(2 * P, 3)
    enc_in8 = jnp.pad(enc_in, ((0, 0), (0, 5)))
    ew1_8 = jnp.pad(enc_W1, ((0, 5), (0, 0)))
    conv_wp = conv_W[:2]
    wx = conv_W[2:]
    w1a = dec_W1[:FEAT]
    w1b = dec_W1[FEAT:]
    w2_8 = jnp.pad(dec_W2, ((0, 0), (0, 7)))
    b2_8 = jnp.pad(dec_b2, ((0, 7)))

    deg = _sc_degree(dst_tiles)

    h, s, dinv, posw = _tc_pass1(enc_in8, ew1_8, enc_b1, enc_W2, enc_b2,
                                 points, pos_pad, deg, conv_wp)
    x = _tc_pass2(points, pos_pad, h, s)

    x, y = _tc_step(x, x, dinv, posw, wx, conv_b, ln_g, ln_b,
                    do_ln=False, do_y=True)
    for t in range(4):
        acc = _sc_spmm(y, sd_tiles)
        x, y = _tc_step(x, acc, dinv, posw, wx, conv_b, ln_g, ln_b,
                        do_ln=True, do_y=(t < 3))

    _, _, out8 = _tc_decoder(qpts, pos_pad, x, w1a, w1b, dec_b1, w2_8, b2_8)
    return out8.reshape(2, 512, 8)[:, :, :1]
